# baseline (speedup 1.0000x reference)
import jax
import jax.numpy as jnp
from jax.experimental import pallas as pl
from jax.experimental.pallas import tpu as pltpu

_B = 512
_R = 4 * 64 * 64
_RB = 1024
_T = 1000


def _body(t_ref, sa_ref, som_ref, x_ref, n_ref, o_ref, ab_s):
    i = pl.program_id(0)

    @pl.when(i == 0)
    def _gather():
        t_row = t_ref[...]
        iota_j = jax.lax.broadcasted_iota(jnp.int32, (_T, _B), 0)
        oh = (iota_j == t_row).astype(jnp.float32)
        tbl = jnp.concatenate([sa_ref[...], som_ref[...]], axis=0)
        ab = jax.lax.dot_general(
            tbl, oh, (((1,), (0,)), ((), ())),
            preferred_element_type=jnp.float32,
            precision=jax.lax.Precision.HIGHEST,
        )
        ab_s[...] = ab

    ab = ab_s[...]
    o_ref[...] = ab[0:1, :] * x_ref[...] + ab[1:2, :] * n_ref[...]


def kernel(x0, t, noise, sqrt_alphas_cumprod, sqrt_one_minus_alphas_cumprod):
    xT = jnp.transpose(x0, (1, 2, 3, 0)).reshape(_R, _B)
    nT = jnp.transpose(noise, (1, 2, 3, 0)).reshape(_R, _B)
    t_row = t.astype(jnp.int32).reshape(1, _B)
    sa_row = sqrt_alphas_cumprod.reshape(1, _T)
    som_row = sqrt_one_minus_alphas_cumprod.reshape(1, _T)
    out = pl.pallas_call(
        _body,
        grid=(_R // _RB,),
        in_specs=[
            pl.BlockSpec((1, _B), lambda i: (0, 0)),
            pl.BlockSpec((1, _T), lambda i: (0, 0)),
            pl.BlockSpec((1, _T), lambda i: (0, 0)),
            pl.BlockSpec((_RB, _B), lambda i: (i, 0)),
            pl.BlockSpec((_RB, _B), lambda i: (i, 0)),
        ],
        out_specs=pl.BlockSpec((_RB, _B), lambda i: (i, 0)),
        out_shape=jax.ShapeDtypeStruct((_R, _B), jnp.float32),
        scratch_shapes=[
            pltpu.VMEM((2, _B), jnp.float32),
        ],
        compiler_params=pltpu.CompilerParams(
            dimension_semantics=("arbitrary",),
        ),
    )(t_row, sa_row, som_row, xT, nT)
    return out.reshape(4, 64, 64, _B).transpose(3, 0, 1, 2)


# speedup vs baseline: 1.0411x; 1.0411x over previous
import jax
import jax.numpy as jnp
from jax.experimental import pallas as pl
from jax.experimental.pallas import tpu as pltpu

_B = 512
_R = 4 * 64 * 64
_RB = 4096
_T = 1000


def _body(t_ref, sa_ref, som_ref, x_ref, n_ref, o_ref, ab_s):
    i = pl.program_id(0)

    @pl.when(i == 0)
    def _gather():
        t_row = t_ref[...]
        iota_j = jax.lax.broadcasted_iota(jnp.int32, (_T, _B), 0)
        oh = (iota_j == t_row).astype(jnp.float32)
        tbl = jnp.concatenate([sa_ref[...], som_ref[...]], axis=0)
        ab = jax.lax.dot_general(
            tbl, oh, (((1,), (0,)), ((), ())),
            preferred_element_type=jnp.float32,
            precision=jax.lax.Precision.HIGHEST,
        )
        ab_s[...] = ab

    ab = ab_s[...]
    o_ref[...] = ab[0:1, :] * x_ref[...] + ab[1:2, :] * n_ref[...]


def kernel(x0, t, noise, sqrt_alphas_cumprod, sqrt_one_minus_alphas_cumprod):
    xT = jnp.transpose(x0, (1, 2, 3, 0)).reshape(_R, _B)
    nT = jnp.transpose(noise, (1, 2, 3, 0)).reshape(_R, _B)
    t_row = t.astype(jnp.int32).reshape(1, _B)
    sa_row = sqrt_alphas_cumprod.reshape(1, _T)
    som_row = sqrt_one_minus_alphas_cumprod.reshape(1, _T)
    out = pl.pallas_call(
        _body,
        grid=(_R // _RB,),
        in_specs=[
            pl.BlockSpec((1, _B), lambda i: (0, 0)),
            pl.BlockSpec((1, _T), lambda i: (0, 0)),
            pl.BlockSpec((1, _T), lambda i: (0, 0)),
            pl.BlockSpec((_RB, _B), lambda i: (i, 0)),
            pl.BlockSpec((_RB, _B), lambda i: (i, 0)),
        ],
        out_specs=pl.BlockSpec((_RB, _B), lambda i: (i, 0)),
        out_shape=jax.ShapeDtypeStruct((_R, _B), jnp.float32),
        scratch_shapes=[
            pltpu.VMEM((2, _B), jnp.float32),
        ],
        compiler_params=pltpu.CompilerParams(
            dimension_semantics=("arbitrary",),
        ),
    )(t_row, sa_row, som_row, xT, nT)
    return out.reshape(4, 64, 64, _B).transpose(3, 0, 1, 2)
